# Initial kernel scaffold; baseline (speedup 1.0000x reference)
import jax
import jax.numpy as jnp
from jax.experimental import pallas as pl


def kernel(x, edge_index, batch, edge_weight, W1, b1, W2, b2, Wf1, bf1, Wf2, bf2):
    raise NotImplementedError("write your pallas kernel here")


# speedup vs baseline: 6.3194x; 6.3194x over previous
import functools

import jax
import jax.numpy as jnp
from jax import lax
from jax.experimental import pallas as pl
from jax.experimental.pallas import tpu as pltpu
from jax.experimental.pallas import tpu_sc as plsc

N = 10000
E = 320000
D = 128
G = 16

NPAD = 10240
NW = 32
EPT = E // NW
K = 128
NCHUNK = (EPT + K - 1) // K
EPT_PAD = NCHUNK * K
ROWS_PER_TILE = NPAD // G
RB = 1024
NBLK = NPAD // RB

_mesh = plsc.VectorSubcoreMesh(core_axis_name="c", subcore_axis_name="s")


@functools.partial(
    pl.kernel,
    out_type=jax.ShapeDtypeStruct((2, NPAD), jnp.float32),
    mesh=_mesh,
    scratch_types=[
        pltpu.VMEM((NCHUNK, K), jnp.int32),
        pltpu.VMEM((NCHUNK, K), jnp.float32),
        pltpu.VMEM((ROWS_PER_TILE,), jnp.float32),
        pltpu.VMEM_SHARED((NPAD,), jnp.float32),
    ],
)
def _sc_degree(dst_hbm, ew_hbm, out_hbm, dst_v, ew_v, zbuf, deg_sh):
    cid = lax.axis_index("c")
    sid = lax.axis_index("s")
    wid = cid * 16 + sid
    zero16 = jnp.zeros((16,), jnp.float32)

    def zfill(i, _):
        zbuf[pl.ds(i * 16, 16)] = zero16
        return _

    lax.fori_loop(0, ROWS_PER_TILE // 16, zfill, None)
    pltpu.sync_copy(zbuf, deg_sh.at[pl.ds(sid * ROWS_PER_TILE, ROWS_PER_TILE)])
    plsc.subcore_barrier()

    pltpu.sync_copy(dst_hbm.at[wid], dst_v)
    pltpu.sync_copy(ew_hbm.at[wid], ew_v)

    def chunk(c, _):
        pltpu.sync_copy(ew_v.at[c], deg_sh.at[dst_v.at[c]], add=True)
        return _

    lax.fori_loop(0, NCHUNK, chunk, None)
    plsc.subcore_barrier()
    pltpu.sync_copy(
        deg_sh.at[pl.ds(sid * ROWS_PER_TILE, ROWS_PER_TILE)],
        out_hbm.at[cid, pl.ds(sid * ROWS_PER_TILE, ROWS_PER_TILE)],
    )


NH = NPAD // 2
ACC_ROWS = NH + 16
EPC = E // G
NCHUNK_E = (EPC + K - 1) // K
EPC_PAD = NCHUNK_E * K
ZROWS = ACC_ROWS // 16
CROWS = NH // 16


@functools.partial(
    pl.kernel,
    out_type=jax.ShapeDtypeStruct((NPAD, D), jnp.float32),
    mesh=_mesh,
    scratch_types=[
        pltpu.VMEM((NCHUNK_E, K), jnp.int32),
        pltpu.VMEM((NCHUNK_E, K), jnp.int32),
        pltpu.VMEM((K,), jnp.int32),
        pltpu.VMEM((K, 16), jnp.float32),
        pltpu.VMEM((K, D), jnp.float32),
        pltpu.SemaphoreType.DMA,
        pltpu.VMEM_SHARED((ACC_ROWS, D), jnp.float32),
    ],
)
def _sc_edge_pass(hs_hbm, src_hbm, dst_hbm, ewb_hbm, out_hbm,
                  src_v, dst_v, dstm_v, ewb_v, msg, gsem, acc_sh):
    cid = lax.axis_index("c")
    sid = lax.axis_index("s")
    base_node = cid * NH
    zero16 = jnp.zeros((16,), jnp.float32)

    def zfill(i, _):
        for j in range(D // 16):
            msg[i, pl.ds(j * 16, 16)] = zero16
        return _

    lax.fori_loop(0, K, zfill, None)
    pltpu.sync_copy(msg, acc_sh.at[pl.ds(sid * ZROWS, K)])
    pltpu.sync_copy(msg, acc_sh.at[pl.ds(sid * ZROWS + K, K)])
    pltpu.sync_copy(msg.at[pl.ds(0, ZROWS - 2 * K)],
                    acc_sh.at[pl.ds(sid * ZROWS + 2 * K, ZROWS - 2 * K)])
    plsc.subcore_barrier()

    pltpu.sync_copy(src_hbm.at[sid], src_v)
    pltpu.sync_copy(dst_hbm.at[sid], dst_v)

    def chunk(c, _):
        pltpu.async_copy(hs_hbm.at[src_v.at[c]], msg, gsem).wait()
        pltpu.sync_copy(ewb_hbm.at[sid, c], ewb_v)

        def scale(i, _):
            w = ewb_v[i, :]
            for j in range(D // 16):
                msg[i, pl.ds(j * 16, 16)] = msg[i, pl.ds(j * 16, 16)] * w
            return _

        lax.fori_loop(0, K, scale, None)
        for g in range(K // 16):
            rel = dst_v[c, pl.ds(g * 16, 16)] - base_node
            ok = jnp.logical_and(rel >= 0, rel < NH)
            dstm_v[pl.ds(g * 16, 16)] = jnp.where(ok, rel, NH)
        pltpu.sync_copy(msg, acc_sh.at[dstm_v], add=True)
        return _

    lax.fori_loop(0, NCHUNK_E, chunk, None)
    plsc.subcore_barrier()
    pltpu.sync_copy(
        acc_sh.at[pl.ds(sid * CROWS, CROWS)],
        out_hbm.at[pl.ds(base_node + sid * CROWS, CROWS)],
    )


def _tc1_body(x_ref, w1_ref, d0_ref, d1_ref, h1_ref, hs_ref, dinv_ref):
    deg = d0_ref[...] + d1_ref[...] + 1.0
    dinv = lax.rsqrt(deg)
    h = jnp.dot(x_ref[...], w1_ref[...], preferred_element_type=jnp.float32)
    h1_ref[...] = h
    hs_ref[...] = h * dinv[:, None]
    dinv_ref[...] = dinv


def _tc2_body(a_ref, h1_ref, dinv_ref, b1_ref, w2_ref, h2_ref, hs_ref):
    dinv = dinv_ref[...]
    z = a_ref[...] * dinv[:, None]
    z = z + h1_ref[...] * (dinv * dinv)[:, None] + b1_ref[...][None, :]
    z = jnp.maximum(z, 0.0)
    h2 = jnp.dot(z, w2_ref[...], preferred_element_type=jnp.float32)
    h2_ref[...] = h2
    hs_ref[...] = h2 * dinv[:, None]


def _tc3_body(a_ref, h2_ref, dinv_ref, b2_ref, batch_ref,
              wf1_ref, bf1_ref, wf2_ref, bf2_ref, out_ref,
              sums_ref, cnts_ref):
    i = pl.program_id(0)

    @pl.when(i == 0)
    def _():
        sums_ref[...] = jnp.zeros_like(sums_ref)
        cnts_ref[...] = jnp.zeros_like(cnts_ref)

    dinv = dinv_ref[...]
    z = a_ref[...] * dinv[:, None]
    z = z + h2_ref[...] * (dinv * dinv)[:, None] + b2_ref[...][None, :]
    z = jnp.maximum(z, 0.0)
    gid = lax.broadcasted_iota(jnp.int32, (G, RB), 0)
    onehot = (batch_ref[...][None, :] == gid).astype(jnp.float32)
    sums_ref[...] += jnp.dot(onehot, z, preferred_element_type=jnp.float32)
    cnts_ref[...] += jnp.dot(onehot, jnp.ones((RB, D), jnp.float32),
                             preferred_element_type=jnp.float32)

    @pl.when(i == NBLK - 1)
    def _():
        pooled = sums_ref[...] / jnp.maximum(cnts_ref[...], 1.0)
        g = jnp.maximum(
            jnp.dot(pooled, wf1_ref[...], preferred_element_type=jnp.float32)
            + bf1_ref[...][None, :], 0.0)
        out_ref[...] = (
            jnp.dot(g, wf2_ref[...], preferred_element_type=jnp.float32)
            + bf2_ref[...][None, :])


_row_spec = pl.BlockSpec((RB, D), lambda i: (i, 0))
_vec_spec = pl.BlockSpec((RB,), lambda i: (i,))
_mat_spec = pl.BlockSpec((D, D), lambda i: (0, 0))
_bias_spec = pl.BlockSpec((D,), lambda i: (0,))

_full_shape = jax.ShapeDtypeStruct((NPAD, D), jnp.float32)

_tc1 = pl.pallas_call(
    _tc1_body,
    grid=(NBLK,),
    in_specs=[_row_spec, _mat_spec, _vec_spec, _vec_spec],
    out_specs=[_row_spec, _row_spec, _vec_spec],
    out_shape=[
        _full_shape, _full_shape,
        jax.ShapeDtypeStruct((NPAD,), jnp.float32),
    ],
)

_tc2 = pl.pallas_call(
    _tc2_body,
    grid=(NBLK,),
    in_specs=[_row_spec, _row_spec, _vec_spec, _bias_spec, _mat_spec],
    out_specs=[_row_spec, _row_spec],
    out_shape=[_full_shape, _full_shape],
)

_tc3 = pl.pallas_call(
    _tc3_body,
    grid=(NBLK,),
    in_specs=[_row_spec, _row_spec, _vec_spec, _bias_spec,
              pl.BlockSpec((RB,), lambda i: (i,)),
              _mat_spec, _bias_spec, _mat_spec, _bias_spec],
    out_specs=pl.BlockSpec((G, D), lambda i: (0, 0)),
    out_shape=jax.ShapeDtypeStruct((G, D), jnp.float32),
    scratch_shapes=[
        pltpu.VMEM((G, D), jnp.float32),
        pltpu.VMEM((G, D), jnp.float32),
    ],
)


def kernel(x, edge_index, batch, edge_weight, W1, b1, W2, b2,
           Wf1, bf1, Wf2, bf2):
    src = edge_index[0]
    dst = edge_index[1]
    ew = edge_weight

    pad32 = EPT_PAD - EPT
    dst3 = jnp.pad(dst.reshape(NW, EPT), ((0, 0), (0, pad32)),
                   constant_values=NPAD - 1).reshape(NW, NCHUNK, K)
    ew3 = jnp.pad(ew.reshape(NW, EPT), ((0, 0), (0, pad32))
                  ).reshape(NW, NCHUNK, K)

    pad16 = EPC_PAD - EPC
    srcE = jnp.pad(src.reshape(G, EPC), ((0, 0), (0, pad16))
                   ).reshape(G, NCHUNK_E, K)
    dstE = jnp.pad(dst.reshape(G, EPC), ((0, 0), (0, pad16)),
                   constant_values=NPAD - 1).reshape(G, NCHUNK_E, K)
    ewE = jnp.pad(ew.reshape(G, EPC), ((0, 0), (0, pad16))
                  ).reshape(G, NCHUNK_E, K)
    ewb = jnp.broadcast_to(ewE[..., None], (G, NCHUNK_E, K, 16))

    xpad = jnp.pad(x, ((0, NPAD - N), (0, 0)))
    batchpad = jnp.pad(batch, (0, NPAD - N), constant_values=G)

    deg2 = _sc_degree(dst3, ew3)
    h1, hs1, dinv = _tc1(xpad, W1, deg2[0], deg2[1])

    a1 = _sc_edge_pass(hs1, srcE, dstE, ewb)
    h2, hs2 = _tc2(a1, h1, dinv, b1, W2)

    a2 = _sc_edge_pass(hs2, srcE, dstE, ewb)
    out = _tc3(a2, h2, dinv, b2, batchpad, Wf1, bf1, Wf2, bf2)
    return out
